# Initial kernel scaffold; baseline (speedup 1.0000x reference)
import jax
import jax.numpy as jnp
from jax.experimental import pallas as pl


def kernel(x, edge_index, ggnn_w, w_ih, w_hh, b_ih, b_hh, ef1_w, ef1_b, ef2_w, ef2_b, ef3_w, ef3_b, cls_w, cls_b):
    raise NotImplementedError("write your pallas kernel here")


# speedup vs baseline: 2.1690x; 2.1690x over previous
import functools

import jax
import jax.numpy as jnp
from jax import lax
from jax.experimental import pallas as pl
from jax.experimental.pallas import tpu as pltpu
from jax.experimental.pallas import tpu_sc as plsc

N = 10000
HID = 200
HP = 256
HH = 128
E = 320000
STEPS = 6
NC = 2
NS = 16
NPASS = 2
EPT = E // NS
CH = 80
SEG = 5
NCHS = EPT // (CH * SEG)
DH = N // NPASS
DHP = 5120
TRASH = 256
ACC = DHP + TRASH
RPT = ACC // NS
WBC = 48


_sc_mesh = plsc.VectorSubcoreMesh(core_axis_name="c", subcore_axis_name="s")


@functools.partial(
    pl.kernel,
    out_type=jax.ShapeDtypeStruct((NC, NPASS, ACC, HH), jnp.float32),
    mesh=_sc_mesh,
    scratch_types=[
        pltpu.VMEM((NCHS, 1, CH), jnp.int32),
        pltpu.VMEM((NCHS, 1, CH), jnp.int32),
        pltpu.VMEM((CH, HH), jnp.float32),
        pltpu.VMEM((WBC, HH), jnp.float32),
        pltpu.VMEM_SHARED((ACC, HH), jnp.float32),
        pltpu.SemaphoreType.DMA,
    ],
)
def _sc_scatter(hsf_hbm, srcs_hbm, dsts_hbm, zrows_hbm, out_hbm,
                src_v, dst_v, rows_v, bounce_v, s_sh, sem):
    cid = lax.axis_index("c")
    sid = lax.axis_index("s")

    for p in range(NPASS):
        pltpu.sync_copy(zrows_hbm, bounce_v)
        for t in range(RPT // WBC):
            pltpu.sync_copy(bounce_v, s_sh.at[pl.ds(sid * RPT + t * WBC, WBC)])

        plsc.subcore_barrier()

        for seg in range(SEG):
            pltpu.sync_copy(srcs_hbm.at[cid, sid, seg], src_v)
            pltpu.sync_copy(dsts_hbm.at[p, sid, seg], dst_v)

            def chunk(j, carry):
                pltpu.async_copy(hsf_hbm.at[src_v.at[j, 0]], rows_v, sem).wait()
                pltpu.sync_copy(rows_v, s_sh.at[dst_v.at[j, 0]], add=True)
                return carry

            lax.fori_loop(0, NCHS, chunk, 0)

        plsc.subcore_barrier()

        for t in range(RPT // WBC):
            pltpu.sync_copy(s_sh.at[pl.ds(sid * RPT + t * WBC, WBC)], bounce_v)
            pltpu.sync_copy(bounce_v,
                            out_hbm.at[cid, p, pl.ds(sid * RPT + t * WBC, WBC)])


RB = 1000
NRB = N // RB


def _gru_body(h_ref, s_ref, w_ref,
              wr_ref, wz_ref, wn_ref, ur_ref, uz_ref, un_ref,
              br_ref, bz_ref, bn_ref, cr_ref, cz_ref, cn_ref,
              out_ref):
    f32 = jnp.float32
    h = h_ref[...]
    s = s_ref[...]
    agg = jnp.dot(s, w_ref[...], preferred_element_type=f32)
    i_r = jnp.dot(agg, wr_ref[...], preferred_element_type=f32) + br_ref[...]
    i_z = jnp.dot(agg, wz_ref[...], preferred_element_type=f32) + bz_ref[...]
    i_n = jnp.dot(agg, wn_ref[...], preferred_element_type=f32) + bn_ref[...]
    h_r = jnp.dot(h, ur_ref[...], preferred_element_type=f32) + cr_ref[...]
    h_z = jnp.dot(h, uz_ref[...], preferred_element_type=f32) + cz_ref[...]
    h_n = jnp.dot(h, un_ref[...], preferred_element_type=f32) + cn_ref[...]
    r = jax.nn.sigmoid(i_r + h_r)
    z = jax.nn.sigmoid(i_z + h_z)
    nn_ = jnp.tanh(i_n + r * h_n)
    out_ref[...] = (1.0 - z) * nn_ + z * h


_row_spec = pl.BlockSpec((RB, HP), lambda i: (i, 0))
_wspec = pl.BlockSpec((HP, HP), lambda i: (0, 0))
_bspec = pl.BlockSpec((1, HP), lambda i: (0, 0))

_gru_step = pl.pallas_call(
    _gru_body,
    grid=(NRB,),
    in_specs=[_row_spec, _row_spec] + [_wspec] * 7 + [_bspec] * 6,
    out_specs=_row_spec,
    out_shape=jax.ShapeDtypeStruct((N, HP), jnp.float32),
)


def _pool_body(h_ref, out_ref):
    @pl.when(pl.program_id(0) == 0)
    def _init():
        out_ref[...] = jnp.zeros_like(out_ref)

    out_ref[...] += jnp.sum(jax.nn.relu(h_ref[...]), axis=0, keepdims=True)


_pool = pl.pallas_call(
    _pool_body,
    grid=(NRB,),
    in_specs=[_row_spec],
    out_specs=pl.BlockSpec((1, HP), lambda i: (0, 0)),
    out_shape=jax.ShapeDtypeStruct((1, HP), jnp.float32),
)


def _mlp_body(p_ref, w1_ref, b1_ref, w2_ref, b2_ref, w3_ref, b3_ref,
              wc_ref, bc_ref, out_ref):
    f32 = jnp.float32
    a = jax.nn.relu(jnp.dot(p_ref[...], w1_ref[...], preferred_element_type=f32) + b1_ref[...])
    a = jax.nn.relu(jnp.dot(a, w2_ref[...], preferred_element_type=f32) + b2_ref[...])
    a = jax.nn.relu(jnp.dot(a, w3_ref[...], preferred_element_type=f32) + b3_ref[...])
    lg = jnp.dot(a, wc_ref[...], preferred_element_type=f32) + bc_ref[...]
    m = jnp.max(lg, axis=-1, keepdims=True)
    e = jnp.exp(lg - m)
    out_ref[...] = e / jnp.sum(e, axis=-1, keepdims=True)


_mlp = pl.pallas_call(
    _mlp_body,
    out_shape=jax.ShapeDtypeStruct((1, 2), jnp.float32),
)


def _pad_w(w):
    return jnp.pad(w, ((0, HP - HID), (0, HP - HID)))


def kernel(x, edge_index, ggnn_w, w_ih, w_hh, b_ih, b_hh,
           ef1_w, ef1_b, ef2_w, ef2_b, ef3_w, ef3_b, cls_w, cls_b):
    h = jnp.pad(x, ((0, 0), (0, HP - x.shape[1])))

    src = edge_index[0].astype(jnp.int32)
    dst = edge_index[1].astype(jnp.int32)
    src_r = src.reshape(NS, SEG, NCHS, 1, CH)
    srcs = jnp.stack([src_r, src_r + N])
    trash = DHP + (jnp.arange(E, dtype=jnp.int32) % TRASH)
    d0 = jnp.where(dst < DH, dst, trash).reshape(NS, SEG, NCHS, 1, CH)
    d1 = jnp.where(dst >= DH, dst - DH, trash).reshape(NS, SEG, NCHS, 1, CH)
    dsts = jnp.stack([d0, d1])
    zrows = jnp.zeros((WBC, HH), dtype=jnp.float32)

    wih_t = w_ih.T
    whh_t = w_hh.T
    wr, wz, wn = (_pad_w(wih_t[:, :HID]), _pad_w(wih_t[:, HID:2 * HID]),
                  _pad_w(wih_t[:, 2 * HID:]))
    ur, uz, un = (_pad_w(whh_t[:, :HID]), _pad_w(whh_t[:, HID:2 * HID]),
                  _pad_w(whh_t[:, 2 * HID:]))

    def _pad_b(b):
        return jnp.pad(b, (0, HP - HID)).reshape(1, HP)

    br, bz, bn = (_pad_b(b_ih[:HID]), _pad_b(b_ih[HID:2 * HID]),
                  _pad_b(b_ih[2 * HID:]))
    cr, cz, cn = (_pad_b(b_hh[:HID]), _pad_b(b_hh[HID:2 * HID]),
                  _pad_b(b_hh[2 * HID:]))

    for i in range(STEPS):
        hsf = jnp.concatenate([h[:, :HH], h[:, HH:]], axis=0)
        q = _sc_scatter(hsf, srcs, dsts, zrows)
        s = jnp.concatenate(
            [jnp.concatenate([q[0, p, :DH], q[1, p, :DH]], axis=1)
             for p in range(NPASS)], axis=0)
        h = _gru_step(h, s, _pad_w(ggnn_w[i]),
                      wr, wz, wn, ur, uz, un, br, bz, bn, cr, cz, cn)

    pooled = _pool(h)
    w1 = jnp.pad(ef1_w.T, ((0, HP - HID), (0, 0)))
    y_a = _mlp(pooled, w1, ef1_b.reshape(1, -1), ef2_w.T,
               ef2_b.reshape(1, -1), ef3_w.T, ef3_b.reshape(1, -1),
               cls_w.T, cls_b.reshape(1, -1))
    return (y_a, x)


# speedup vs baseline: 6.5797x; 3.0335x over previous
import functools

import jax
import jax.numpy as jnp
from jax import lax
from jax.experimental import pallas as pl
from jax.experimental.pallas import tpu as pltpu
from jax.experimental.pallas import tpu_sc as plsc

N = 10000
HID = 200
HP = 256
HH = 128
E = 320000
STEPS = 6
NC = 2
NS = 16
EPT = E // NS
CH = 100
SEG = 10
NCHS = EPT // (CH * SEG)
NPAIR = NCHS // 2
ACC = 10240
RPT = ACC // NS
WB = 80


_sc_mesh = plsc.VectorSubcoreMesh(core_axis_name="c", subcore_axis_name="s")


@functools.partial(
    pl.kernel,
    out_type=jax.ShapeDtypeStruct((NC, ACC, HH), jnp.float32),
    mesh=_sc_mesh,
    scratch_types=[
        pltpu.VMEM((NCHS, 1, CH), jnp.int32),
        pltpu.VMEM((NCHS, 1, CH), jnp.int32),
        pltpu.VMEM((CH, HH), jnp.float32),
        pltpu.VMEM((CH, HH), jnp.float32),
        pltpu.VMEM_SHARED((ACC, HH), jnp.float32),
        pltpu.SemaphoreType.DMA,
        pltpu.SemaphoreType.DMA,
    ],
)
def _sc_scatter(hsf_hbm, srcs_hbm, dsts_hbm, zrows_hbm, out_hbm,
                src_v, dst_v, rows0, rows1, s_sh, sem0, sem1):
    cid = lax.axis_index("c")
    sid = lax.axis_index("s")

    pltpu.sync_copy(zrows_hbm, rows0.at[pl.ds(0, WB)])
    for t in range(RPT // WB):
        pltpu.sync_copy(rows0.at[pl.ds(0, WB)],
                        s_sh.at[pl.ds(sid * RPT + t * WB, WB)])

    plsc.subcore_barrier()

    for seg in range(SEG):
        pltpu.sync_copy(srcs_hbm.at[cid, sid, seg], src_v)
        pltpu.sync_copy(dsts_hbm.at[sid, seg], dst_v)

        pltpu.async_copy(hsf_hbm.at[src_v.at[0, 0]], rows0, sem0)
        pltpu.async_copy(hsf_hbm.at[src_v.at[1, 0]], rows1, sem1)

        def pair(i, carry):
            j0 = 2 * i
            pltpu.make_async_copy(hsf_hbm.at[src_v.at[j0, 0]], rows0,
                                  sem0).wait()
            pltpu.sync_copy(rows0, s_sh.at[dst_v.at[j0, 0]], add=True)

            @pl.when(i < NPAIR - 1)
            def _():
                pltpu.async_copy(hsf_hbm.at[src_v.at[j0 + 2, 0]], rows0, sem0)

            pltpu.make_async_copy(hsf_hbm.at[src_v.at[j0 + 1, 0]], rows1,
                                  sem1).wait()
            pltpu.sync_copy(rows1, s_sh.at[dst_v.at[j0 + 1, 0]], add=True)

            @pl.when(i < NPAIR - 1)
            def _():
                pltpu.async_copy(hsf_hbm.at[src_v.at[j0 + 3, 0]], rows1, sem1)

            return carry

        lax.fori_loop(0, NPAIR, pair, 0)

    plsc.subcore_barrier()

    for t in range(RPT // WB):
        pltpu.sync_copy(s_sh.at[pl.ds(sid * RPT + t * WB, WB)],
                        rows0.at[pl.ds(0, WB)])
        pltpu.sync_copy(rows0.at[pl.ds(0, WB)],
                        out_hbm.at[cid, pl.ds(sid * RPT + t * WB, WB)])


RB = 1000
NRB = N // RB


def _gru_body(h_ref, s_ref, w_ref,
              wr_ref, wz_ref, wn_ref, ur_ref, uz_ref, un_ref,
              br_ref, bz_ref, bn_ref, cr_ref, cz_ref, cn_ref,
              out_ref):
    f32 = jnp.float32
    h = h_ref[...]
    s = s_ref[...]
    agg = jnp.dot(s, w_ref[...], preferred_element_type=f32)
    i_r = jnp.dot(agg, wr_ref[...], preferred_element_type=f32) + br_ref[...]
    i_z = jnp.dot(agg, wz_ref[...], preferred_element_type=f32) + bz_ref[...]
    i_n = jnp.dot(agg, wn_ref[...], preferred_element_type=f32) + bn_ref[...]
    h_r = jnp.dot(h, ur_ref[...], preferred_element_type=f32) + cr_ref[...]
    h_z = jnp.dot(h, uz_ref[...], preferred_element_type=f32) + cz_ref[...]
    h_n = jnp.dot(h, un_ref[...], preferred_element_type=f32) + cn_ref[...]
    r = jax.nn.sigmoid(i_r + h_r)
    z = jax.nn.sigmoid(i_z + h_z)
    nn_ = jnp.tanh(i_n + r * h_n)
    out_ref[...] = (1.0 - z) * nn_ + z * h


_row_spec = pl.BlockSpec((RB, HP), lambda i: (i, 0))
_wspec = pl.BlockSpec((HP, HP), lambda i: (0, 0))
_bspec = pl.BlockSpec((1, HP), lambda i: (0, 0))

_gru_step = pl.pallas_call(
    _gru_body,
    grid=(NRB,),
    in_specs=[_row_spec, _row_spec] + [_wspec] * 7 + [_bspec] * 6,
    out_specs=_row_spec,
    out_shape=jax.ShapeDtypeStruct((N, HP), jnp.float32),
)


def _pool_body(h_ref, out_ref):
    @pl.when(pl.program_id(0) == 0)
    def _init():
        out_ref[...] = jnp.zeros_like(out_ref)

    out_ref[...] += jnp.sum(jax.nn.relu(h_ref[...]), axis=0, keepdims=True)


_pool = pl.pallas_call(
    _pool_body,
    grid=(NRB,),
    in_specs=[_row_spec],
    out_specs=pl.BlockSpec((1, HP), lambda i: (0, 0)),
    out_shape=jax.ShapeDtypeStruct((1, HP), jnp.float32),
)


def _mlp_body(p_ref, w1_ref, b1_ref, w2_ref, b2_ref, w3_ref, b3_ref,
              wc_ref, bc_ref, out_ref):
    f32 = jnp.float32
    a = jax.nn.relu(jnp.dot(p_ref[...], w1_ref[...], preferred_element_type=f32) + b1_ref[...])
    a = jax.nn.relu(jnp.dot(a, w2_ref[...], preferred_element_type=f32) + b2_ref[...])
    a = jax.nn.relu(jnp.dot(a, w3_ref[...], preferred_element_type=f32) + b3_ref[...])
    lg = jnp.dot(a, wc_ref[...], preferred_element_type=f32) + bc_ref[...]
    m = jnp.max(lg, axis=-1, keepdims=True)
    e = jnp.exp(lg - m)
    out_ref[...] = e / jnp.sum(e, axis=-1, keepdims=True)


_mlp = pl.pallas_call(
    _mlp_body,
    out_shape=jax.ShapeDtypeStruct((1, 2), jnp.float32),
)


def _pad_w(w):
    return jnp.pad(w, ((0, HP - HID), (0, HP - HID)))


def kernel(x, edge_index, ggnn_w, w_ih, w_hh, b_ih, b_hh,
           ef1_w, ef1_b, ef2_w, ef2_b, ef3_w, ef3_b, cls_w, cls_b):
    h = jnp.pad(x, ((0, 0), (0, HP - x.shape[1])))

    src = edge_index[0].astype(jnp.int32)
    dst = edge_index[1].astype(jnp.int32)
    src_r = src.reshape(NS, SEG, NCHS, 1, CH)
    srcs = jnp.stack([src_r, src_r + N])
    dsts = dst.reshape(NS, SEG, NCHS, 1, CH)
    zrows = jnp.zeros((WB, HH), dtype=jnp.float32)

    wih_t = w_ih.T
    whh_t = w_hh.T
    wr, wz, wn = (_pad_w(wih_t[:, :HID]), _pad_w(wih_t[:, HID:2 * HID]),
                  _pad_w(wih_t[:, 2 * HID:]))
    ur, uz, un = (_pad_w(whh_t[:, :HID]), _pad_w(whh_t[:, HID:2 * HID]),
                  _pad_w(whh_t[:, 2 * HID:]))

    def _pad_b(b):
        return jnp.pad(b, (0, HP - HID)).reshape(1, HP)

    br, bz, bn = (_pad_b(b_ih[:HID]), _pad_b(b_ih[HID:2 * HID]),
                  _pad_b(b_ih[2 * HID:]))
    cr, cz, cn = (_pad_b(b_hh[:HID]), _pad_b(b_hh[HID:2 * HID]),
                  _pad_b(b_hh[2 * HID:]))

    for i in range(STEPS):
        hsf = jnp.concatenate([h[:, :HH], h[:, HH:]], axis=0)
        q = _sc_scatter(hsf, srcs, dsts, zrows)
        s = jnp.concatenate([q[0, :N], q[1, :N]], axis=1)
        h = _gru_step(h, s, _pad_w(ggnn_w[i]),
                      wr, wz, wn, ur, uz, un, br, bz, bn, cr, cz, cn)

    pooled = _pool(h)
    w1 = jnp.pad(ef1_w.T, ((0, HP - HID), (0, 0)))
    y_a = _mlp(pooled, w1, ef1_b.reshape(1, -1), ef2_w.T,
               ef2_b.reshape(1, -1), ef3_w.T, ef3_b.reshape(1, -1),
               cls_w.T, cls_b.reshape(1, -1))
    return (y_a, x)


# speedup vs baseline: 6.9194x; 1.0516x over previous
import functools

import jax
import jax.numpy as jnp
from jax import lax
from jax.experimental import pallas as pl
from jax.experimental.pallas import tpu as pltpu
from jax.experimental.pallas import tpu_sc as plsc

N = 10000
HID = 200
HP = 256
HH = 128
E = 320000
STEPS = 6
NC = 2
NS = 16
EPT = E // NS
CH = 125
SEG = 8
NCHS = EPT // (CH * SEG)
NPAIR = NCHS // 2
ACC = 10240
RPT = ACC // NS
WB = 80


_sc_mesh = plsc.VectorSubcoreMesh(core_axis_name="c", subcore_axis_name="s")


@functools.partial(
    pl.kernel,
    out_type=jax.ShapeDtypeStruct((NC, ACC, HH), jnp.float32),
    mesh=_sc_mesh,
    scratch_types=[
        pltpu.VMEM((NCHS, 1, CH), jnp.int32),
        pltpu.VMEM((NCHS, 1, CH), jnp.int32),
        pltpu.VMEM((CH, HH), jnp.float32),
        pltpu.VMEM((CH, HH), jnp.float32),
        pltpu.VMEM_SHARED((ACC, HH), jnp.float32),
        pltpu.SemaphoreType.DMA,
        pltpu.SemaphoreType.DMA,
    ],
)
def _sc_scatter(hsf_hbm, srcs_hbm, dsts_hbm, zrows_hbm, out_hbm,
                src_v, dst_v, rows0, rows1, s_sh, sem0, sem1):
    cid = lax.axis_index("c")
    sid = lax.axis_index("s")

    pltpu.sync_copy(zrows_hbm, rows0.at[pl.ds(0, WB)])
    for t in range(RPT // WB):
        pltpu.sync_copy(rows0.at[pl.ds(0, WB)],
                        s_sh.at[pl.ds(sid * RPT + t * WB, WB)])

    plsc.subcore_barrier()

    for seg in range(SEG):
        pltpu.sync_copy(srcs_hbm.at[cid, sid, seg], src_v)
        pltpu.sync_copy(dsts_hbm.at[sid, seg], dst_v)

        pltpu.async_copy(hsf_hbm.at[src_v.at[0, 0]], rows0, sem0)
        pltpu.async_copy(hsf_hbm.at[src_v.at[1, 0]], rows1, sem1)

        def pair(i, carry):
            j0 = 2 * i
            pltpu.make_async_copy(hsf_hbm.at[src_v.at[j0, 0]], rows0,
                                  sem0).wait()
            pltpu.sync_copy(rows0, s_sh.at[dst_v.at[j0, 0]], add=True)

            @pl.when(i < NPAIR - 1)
            def _():
                pltpu.async_copy(hsf_hbm.at[src_v.at[j0 + 2, 0]], rows0, sem0)

            pltpu.make_async_copy(hsf_hbm.at[src_v.at[j0 + 1, 0]], rows1,
                                  sem1).wait()
            pltpu.sync_copy(rows1, s_sh.at[dst_v.at[j0 + 1, 0]], add=True)

            @pl.when(i < NPAIR - 1)
            def _():
                pltpu.async_copy(hsf_hbm.at[src_v.at[j0 + 3, 0]], rows1, sem1)

            return carry

        lax.fori_loop(0, NPAIR, pair, 0)

    plsc.subcore_barrier()

    for t in range(RPT // WB):
        pltpu.sync_copy(s_sh.at[pl.ds(sid * RPT + t * WB, WB)],
                        rows0.at[pl.ds(0, WB)])
        pltpu.sync_copy(rows0.at[pl.ds(0, WB)],
                        out_hbm.at[cid, pl.ds(sid * RPT + t * WB, WB)])


RB = 1000
NRB = N // RB


def _gru_body(h_ref, s_ref, w_ref,
              wr_ref, wz_ref, wn_ref, ur_ref, uz_ref, un_ref,
              br_ref, bz_ref, bn_ref, cr_ref, cz_ref, cn_ref,
              out_ref):
    f32 = jnp.float32
    h = h_ref[...]
    s = s_ref[...]
    agg = jnp.dot(s, w_ref[...], preferred_element_type=f32)
    i_r = jnp.dot(agg, wr_ref[...], preferred_element_type=f32) + br_ref[...]
    i_z = jnp.dot(agg, wz_ref[...], preferred_element_type=f32) + bz_ref[...]
    i_n = jnp.dot(agg, wn_ref[...], preferred_element_type=f32) + bn_ref[...]
    h_r = jnp.dot(h, ur_ref[...], preferred_element_type=f32) + cr_ref[...]
    h_z = jnp.dot(h, uz_ref[...], preferred_element_type=f32) + cz_ref[...]
    h_n = jnp.dot(h, un_ref[...], preferred_element_type=f32) + cn_ref[...]
    r = jax.nn.sigmoid(i_r + h_r)
    z = jax.nn.sigmoid(i_z + h_z)
    nn_ = jnp.tanh(i_n + r * h_n)
    out_ref[...] = (1.0 - z) * nn_ + z * h


_row_spec = pl.BlockSpec((RB, HP), lambda i: (i, 0))
_wspec = pl.BlockSpec((HP, HP), lambda i: (0, 0))
_bspec = pl.BlockSpec((1, HP), lambda i: (0, 0))

_gru_step = pl.pallas_call(
    _gru_body,
    grid=(NRB,),
    in_specs=[_row_spec, _row_spec] + [_wspec] * 7 + [_bspec] * 6,
    out_specs=_row_spec,
    out_shape=jax.ShapeDtypeStruct((N, HP), jnp.float32),
)


def _pool_body(h_ref, out_ref):
    @pl.when(pl.program_id(0) == 0)
    def _init():
        out_ref[...] = jnp.zeros_like(out_ref)

    out_ref[...] += jnp.sum(jax.nn.relu(h_ref[...]), axis=0, keepdims=True)


_pool = pl.pallas_call(
    _pool_body,
    grid=(NRB,),
    in_specs=[_row_spec],
    out_specs=pl.BlockSpec((1, HP), lambda i: (0, 0)),
    out_shape=jax.ShapeDtypeStruct((1, HP), jnp.float32),
)


def _mlp_body(p_ref, w1_ref, b1_ref, w2_ref, b2_ref, w3_ref, b3_ref,
              wc_ref, bc_ref, out_ref):
    f32 = jnp.float32
    a = jax.nn.relu(jnp.dot(p_ref[...], w1_ref[...], preferred_element_type=f32) + b1_ref[...])
    a = jax.nn.relu(jnp.dot(a, w2_ref[...], preferred_element_type=f32) + b2_ref[...])
    a = jax.nn.relu(jnp.dot(a, w3_ref[...], preferred_element_type=f32) + b3_ref[...])
    lg = jnp.dot(a, wc_ref[...], preferred_element_type=f32) + bc_ref[...]
    m = jnp.max(lg, axis=-1, keepdims=True)
    e = jnp.exp(lg - m)
    out_ref[...] = e / jnp.sum(e, axis=-1, keepdims=True)


_mlp = pl.pallas_call(
    _mlp_body,
    out_shape=jax.ShapeDtypeStruct((1, 2), jnp.float32),
)


def _pad_w(w):
    return jnp.pad(w, ((0, HP - HID), (0, HP - HID)))


def kernel(x, edge_index, ggnn_w, w_ih, w_hh, b_ih, b_hh,
           ef1_w, ef1_b, ef2_w, ef2_b, ef3_w, ef3_b, cls_w, cls_b):
    h = jnp.pad(x, ((0, 0), (0, HP - x.shape[1])))

    src = edge_index[0].astype(jnp.int32)
    dst = edge_index[1].astype(jnp.int32)
    src_r = src.reshape(NS, SEG, NCHS, 1, CH)
    srcs = jnp.stack([src_r, src_r + N])
    dsts = dst.reshape(NS, SEG, NCHS, 1, CH)
    zrows = jnp.zeros((WB, HH), dtype=jnp.float32)

    wih_t = w_ih.T
    whh_t = w_hh.T
    wr, wz, wn = (_pad_w(wih_t[:, :HID]), _pad_w(wih_t[:, HID:2 * HID]),
                  _pad_w(wih_t[:, 2 * HID:]))
    ur, uz, un = (_pad_w(whh_t[:, :HID]), _pad_w(whh_t[:, HID:2 * HID]),
                  _pad_w(whh_t[:, 2 * HID:]))

    def _pad_b(b):
        return jnp.pad(b, (0, HP - HID)).reshape(1, HP)

    br, bz, bn = (_pad_b(b_ih[:HID]), _pad_b(b_ih[HID:2 * HID]),
                  _pad_b(b_ih[2 * HID:]))
    cr, cz, cn = (_pad_b(b_hh[:HID]), _pad_b(b_hh[HID:2 * HID]),
                  _pad_b(b_hh[2 * HID:]))

    for i in range(STEPS):
        hsf = jnp.concatenate([h[:, :HH], h[:, HH:]], axis=0)
        q = _sc_scatter(hsf, srcs, dsts, zrows)
        s = jnp.concatenate([q[0, :N], q[1, :N]], axis=1)
        h = _gru_step(h, s, _pad_w(ggnn_w[i]),
                      wr, wz, wn, ur, uz, un, br, bz, bn, cr, cz, cn)

    pooled = _pool(h)
    w1 = jnp.pad(ef1_w.T, ((0, HP - HID), (0, 0)))
    y_a = _mlp(pooled, w1, ef1_b.reshape(1, -1), ef2_w.T,
               ef2_b.reshape(1, -1), ef3_w.T, ef3_b.reshape(1, -1),
               cls_w.T, cls_b.reshape(1, -1))
    return (y_a, x)
